# baseline (speedup 1.0000x reference)
import jax
import jax.numpy as jnp
from jax import lax
from jax.experimental import pallas as pl
from jax.experimental.pallas import tpu as pltpu
from jax.experimental.pallas import tpu_sc as plsc

N_NODES = 10000
N_EDGES = 320000
D_NODE = 128
D_EDGE = 16

NC = 2
NS = 16
NW = NC * NS
EDGES_PER_BLK = 128
BPG = 10
GROUPS = 8
BLKS_PER_TILE = BPG * GROUPS
EDGES_PER_TILE = EDGES_PER_BLK * BLKS_PER_TILE
E_PAD = NW * EDGES_PER_TILE
PACK = 128 // D_EDGE
ROWS = N_EDGES // PACK
GRP_EDGES = BPG * EDGES_PER_BLK
GRP_ROWS = GRP_EDGES // PACK
TILE_ROWS = GROUPS * GRP_ROWS


def _pq_body(ns_ref, wa_ref, wb_ref, p_ref, q_ref):
    ns = ns_ref[...]
    p_ref[...] = jnp.dot(ns, wa_ref[...], preferred_element_type=jnp.float32)
    q_ref[...] = jnp.dot(ns, wb_ref[...], preferred_element_type=jnp.float32)


def _precompute_pq(ns, wa, wb):
    br = 2000
    return pl.pallas_call(
        _pq_body,
        grid=(N_NODES // br,),
        in_specs=[
            pl.BlockSpec((br, D_NODE), lambda t: (t, 0)),
            pl.BlockSpec((D_NODE, D_EDGE), lambda t: (0, 0)),
            pl.BlockSpec((D_NODE, D_EDGE), lambda t: (0, 0)),
        ],
        out_specs=[
            pl.BlockSpec((br, D_EDGE), lambda t: (t, 0)),
            pl.BlockSpec((br, D_EDGE), lambda t: (t, 0)),
        ],
        out_shape=[jax.ShapeDtypeStruct((N_NODES, D_EDGE), jnp.float32)] * 2,
    )(ns, wa, wb)


def _gather_body(p_hbm, q_hbm, sidx_hbm, didx_hbm, gs_hbm, gd_hbm,
                 sidx_v, didx_v, gs0, gs1, gd0, gd1, gsem, wsem):
    wid = lax.axis_index("s") * NC + lax.axis_index("c")
    pltpu.sync_copy(sidx_hbm.at[wid], sidx_v)
    pltpu.sync_copy(didx_hbm.at[wid], didx_v)
    base = wid * EDGES_PER_TILE
    gsb = (gs0, gs1)
    gdb = (gd0, gd1)

    def g_pairs(g, par):
        idx_s = sidx_v.at[pl.ds(g * GRP_EDGES, GRP_EDGES)]
        idx_d = didx_v.at[pl.ds(g * GRP_EDGES, GRP_EDGES)]
        return ((p_hbm.at[idx_s], gsb[par], gsem),
                (q_hbm.at[idx_d], gdb[par], gsem))

    def w_pairs(g, par):
        e0 = base + g * GRP_EDGES
        return ((gsb[par], gs_hbm.at[pl.ds(e0, GRP_EDGES)], wsem),
                (gdb[par], gd_hbm.at[pl.ds(e0, GRP_EDGES)], wsem))

    def fire(pairs):
        for s, d, sem in pairs:
            pltpu.async_copy(s, d, sem)

    def drain(pairs):
        for s, d, sem in pairs:
            pltpu.make_async_copy(s, d, sem).wait()

    fire(g_pairs(0, 0))
    for g in range(GROUPS):
        par = g & 1
        if g + 1 < GROUPS:
            if g >= 1:
                drain(w_pairs(g - 1, (g + 1) & 1))
            fire(g_pairs(g + 1, (g + 1) & 1))
        drain(g_pairs(g, par))
        fire(w_pairs(g, par))
    drain(w_pairs(GROUPS - 2, (GROUPS - 2) & 1))
    drain(w_pairs(GROUPS - 1, (GROUPS - 1) & 1))


def _gather(p, q, sidx, didx):
    mesh = plsc.VectorSubcoreMesh(core_axis_name="c", subcore_axis_name="s")
    out = jax.ShapeDtypeStruct((E_PAD, D_EDGE), jnp.float32)
    buf = pltpu.VMEM((GRP_EDGES, D_EDGE), jnp.float32)
    f = pl.kernel(
        _gather_body,
        out_type=[out, out],
        mesh=mesh,
        scratch_types=[
            pltpu.VMEM((EDGES_PER_TILE,), jnp.int32),
            pltpu.VMEM((EDGES_PER_TILE,), jnp.int32),
            buf, buf, buf, buf,
            pltpu.SemaphoreType.DMA,
            pltpu.SemaphoreType.DMA,
        ],
        compiler_params=pltpu.CompilerParams(use_tc_tiling_on_sc=False),
    )
    return f(p, q, sidx, didx)


def _dense_body(gs_ref, gd_ref, e_ref, w1_ref, w2_ref, ma_ref, pr_ref, o_ref):
    e = e_ref[...]
    x = (gs_ref[...] + gd_ref[...]
         + jnp.dot(e, w1_ref[...], preferred_element_type=jnp.float32)
         + pr_ref[0:1, :])
    h1 = x * (1.0 / (1.0 + jnp.exp(-x)))
    y = jnp.dot(h1, w2_ref[...], preferred_element_type=jnp.float32) + pr_ref[1:2, :]
    h2 = y * (1.0 / (1.0 + jnp.exp(-y)))
    z = e + h2
    m = jnp.dot(z, ma_ref[...], preferred_element_type=jnp.float32)
    s2 = jnp.dot(z * z, ma_ref[...], preferred_element_type=jnp.float32)
    var = s2 - m * m
    o_ref[...] = (z - m) * lax.rsqrt(var + 1e-5) * pr_ref[2:3, :] + pr_ref[3:4, :]


def _dense(gs_pk, gd_pk, e_pk, w1blk, w2blk, mavg, params):
    br = 2000
    full = lambda t: (0, 0)
    row = lambda t: (t, 0)
    return pl.pallas_call(
        _dense_body,
        grid=(ROWS // br,),
        in_specs=[
            pl.BlockSpec((br, 128), row),
            pl.BlockSpec((br, 128), row),
            pl.BlockSpec((br, 128), row),
            pl.BlockSpec((128, 128), full),
            pl.BlockSpec((128, 128), full),
            pl.BlockSpec((128, 128), full),
            pl.BlockSpec((8, 128), full),
        ],
        out_specs=pl.BlockSpec((br, 128), row),
        out_shape=jax.ShapeDtypeStruct((ROWS, 128), jnp.float32),
    )(gs_pk, gd_pk, e_pk, w1blk, w2blk, mavg, params)


def kernel(node_scalars, edge_index, edge_feats, W1, b1, W2, b2, gamma, beta):
    wa = W1[:D_NODE]
    wb = W1[D_NODE:2 * D_NODE]
    we = W1[2 * D_NODE:]

    p, q = _precompute_pq(node_scalars, wa, wb)

    pad = E_PAD - N_EDGES
    src = jnp.pad(edge_index[0].astype(jnp.int32), (0, pad))
    dst = jnp.pad(edge_index[1].astype(jnp.int32), (0, pad))
    sidx = src.reshape(NW, EDGES_PER_TILE)
    didx = dst.reshape(NW, EDGES_PER_TILE)

    gs, gd = _gather(p, q, sidx, didx)
    gs_pk = gs.reshape(-1, 128)
    gd_pk = gd.reshape(-1, 128)
    e_pk = edge_feats.reshape(ROWS, 128)

    eye = jnp.eye(PACK, dtype=jnp.float32)
    w1blk = jnp.kron(eye, we)
    w2blk = jnp.kron(eye, W2)
    mavg = jnp.kron(eye, jnp.full((D_EDGE, D_EDGE), 1.0 / D_EDGE, jnp.float32))
    params = jnp.concatenate([
        jnp.tile(b1, PACK)[None],
        jnp.tile(b2, PACK)[None],
        jnp.tile(gamma, PACK)[None],
        jnp.tile(beta, PACK)[None],
        jnp.zeros((4, 128), jnp.float32),
    ], axis=0)

    out_pk = _dense(gs_pk, gd_pk, e_pk, w1blk, w2blk, mavg, params)
    return out_pk.reshape(N_EDGES, D_EDGE)


# speedup vs baseline: 1.0495x; 1.0495x over previous
import jax
import jax.numpy as jnp
from jax import lax
from jax.experimental import pallas as pl
from jax.experimental.pallas import tpu as pltpu
from jax.experimental.pallas import tpu_sc as plsc

N_NODES = 10000
N_EDGES = 320000
D_NODE = 128
D_EDGE = 16

NC = 2
NS = 16
NW = NC * NS
GRP_EDGES = 1024
GROUPS = 10
EDGES_PER_TILE = GRP_EDGES * GROUPS
E_PAD = NW * EDGES_PER_TILE
PACK = 128 // D_EDGE
ROWS = N_EDGES // PACK


def _pq_body(ns_ref, wa_ref, wb_ref, t_ref):
    half = pl.num_programs(0) // 2
    t = pl.program_id(0)
    w = jnp.where(t < half, wa_ref[...], wb_ref[...])
    t_ref[...] = jnp.dot(ns_ref[...], w, preferred_element_type=jnp.float32)


def _precompute_table(ns, wa, wb):
    br = 2000
    nb = N_NODES // br
    return pl.pallas_call(
        _pq_body,
        grid=(2 * nb,),
        in_specs=[
            pl.BlockSpec((br, D_NODE), lambda t: (t % (N_NODES // 2000), 0)),
            pl.BlockSpec((D_NODE, D_EDGE), lambda t: (0, 0)),
            pl.BlockSpec((D_NODE, D_EDGE), lambda t: (0, 0)),
        ],
        out_specs=pl.BlockSpec((br, D_EDGE), lambda t: (t, 0)),
        out_shape=jax.ShapeDtypeStruct((2 * N_NODES, D_EDGE), jnp.float32),
    )(ns, wa, wb)


def _gather_body(t_hbm, cidx_hbm, s_hbm,
                 idx_v, a0, a1, b0, b1, c0, c1, gsem, wsem):
    wid = lax.axis_index("s") * NC + lax.axis_index("c")
    pltpu.sync_copy(cidx_hbm.at[wid], idx_v)
    base = wid * EDGES_PER_TILE
    ab = ((a0, b0), (a1, b1))
    cb = (c0, c1)

    def g_pairs(g, par):
        sl = pl.ds(g * GRP_EDGES, GRP_EDGES)
        return ((t_hbm.at[idx_v.at[0, sl]], ab[par][0], gsem),
                (t_hbm.at[idx_v.at[1, sl]], ab[par][1], gsem))

    def w_pair(g, par):
        e0 = base + g * GRP_EDGES
        return ((cb[par], s_hbm.at[pl.ds(e0, GRP_EDGES)], wsem),)

    def fire(pairs):
        for s, d, sem in pairs:
            pltpu.async_copy(s, d, sem)

    def drain(pairs):
        for s, d, sem in pairs:
            pltpu.make_async_copy(s, d, sem).wait()

    fire(g_pairs(0, 0))
    for g in range(GROUPS):
        par = g & 1
        if g + 1 < GROUPS:
            fire(g_pairs(g + 1, 1 - par))
        drain(g_pairs(g, par))
        if g >= 2:
            drain(w_pair(g - 2, par))
        a, b = ab[par]
        c = cb[par]

        @pl.loop(0, GRP_EDGES, unroll=8)
        def _add(i):
            c[i, :] = a[i, :] + b[i, :]

        fire(w_pair(g, par))
    drain(w_pair(GROUPS - 2, (GROUPS - 2) & 1))
    drain(w_pair(GROUPS - 1, (GROUPS - 1) & 1))


def _gather_add(table, cidx):
    mesh = plsc.VectorSubcoreMesh(core_axis_name="c", subcore_axis_name="s")
    out = jax.ShapeDtypeStruct((E_PAD, D_EDGE), jnp.float32)
    buf = pltpu.VMEM((GRP_EDGES, D_EDGE), jnp.float32)
    f = pl.kernel(
        _gather_body,
        out_type=out,
        mesh=mesh,
        scratch_types=[
            pltpu.VMEM((2, EDGES_PER_TILE), jnp.int32),
            buf, buf, buf, buf, buf, buf,
            pltpu.SemaphoreType.DMA,
            pltpu.SemaphoreType.DMA,
        ],
        compiler_params=pltpu.CompilerParams(use_tc_tiling_on_sc=False),
    )
    return f(table, cidx)


def _dense_body(s_ref, e_ref, w1_ref, w2_ref, ma_ref, pr_ref, o_ref):
    e = e_ref[...]
    x = (s_ref[...]
         + jnp.dot(e, w1_ref[...], preferred_element_type=jnp.float32)
         + pr_ref[0:1, :])
    h1 = x * (1.0 / (1.0 + jnp.exp(-x)))
    y = jnp.dot(h1, w2_ref[...], preferred_element_type=jnp.float32) + pr_ref[1:2, :]
    h2 = y * (1.0 / (1.0 + jnp.exp(-y)))
    z = e + h2
    m = jnp.dot(z, ma_ref[...], preferred_element_type=jnp.float32)
    s2 = jnp.dot(z * z, ma_ref[...], preferred_element_type=jnp.float32)
    var = s2 - m * m
    o_ref[...] = (z - m) * lax.rsqrt(var + 1e-5) * pr_ref[2:3, :] + pr_ref[3:4, :]


def _dense(s_pk, e_pk, w1blk, w2blk, mavg, params):
    br = 2000
    full = lambda t: (0, 0)
    row = lambda t: (t, 0)
    return pl.pallas_call(
        _dense_body,
        grid=(ROWS // br,),
        in_specs=[
            pl.BlockSpec((br, 128), row),
            pl.BlockSpec((br, 128), row),
            pl.BlockSpec((128, 128), full),
            pl.BlockSpec((128, 128), full),
            pl.BlockSpec((128, 128), full),
            pl.BlockSpec((8, 128), full),
        ],
        out_specs=pl.BlockSpec((br, 128), row),
        out_shape=jax.ShapeDtypeStruct((ROWS, 128), jnp.float32),
    )(s_pk, e_pk, w1blk, w2blk, mavg, params)


def kernel(node_scalars, edge_index, edge_feats, W1, b1, W2, b2, gamma, beta):
    wa = W1[:D_NODE]
    wb = W1[D_NODE:2 * D_NODE]
    we = W1[2 * D_NODE:]

    table = _precompute_table(node_scalars, wa, wb)

    pad = E_PAD - N_EDGES
    src = jnp.pad(edge_index[0].astype(jnp.int32), (0, pad))
    dst = jnp.pad(edge_index[1].astype(jnp.int32), (0, pad)) + N_NODES
    cidx = jnp.stack([src.reshape(NW, EDGES_PER_TILE),
                      dst.reshape(NW, EDGES_PER_TILE)], axis=1)

    s = _gather_add(table, cidx)
    s_pk = s.reshape(-1, 128)
    e_pk = edge_feats.reshape(ROWS, 128)

    eye = jnp.eye(PACK, dtype=jnp.float32)
    w1blk = jnp.kron(eye, we)
    w2blk = jnp.kron(eye, W2)
    mavg = jnp.kron(eye, jnp.full((D_EDGE, D_EDGE), 1.0 / D_EDGE, jnp.float32))
    params = jnp.concatenate([
        jnp.tile(b1, PACK)[None],
        jnp.tile(b2, PACK)[None],
        jnp.tile(gamma, PACK)[None],
        jnp.tile(beta, PACK)[None],
        jnp.zeros((4, 128), jnp.float32),
    ], axis=0)

    out_pk = _dense(s_pk, e_pk, w1blk, w2blk, mavg, params)
    return out_pk.reshape(N_EDGES, D_EDGE)
